# baseline (speedup 1.0000x reference)
import functools

import jax
import jax.numpy as jnp
import numpy as np
from jax.experimental import pallas as pl
from jax.experimental.pallas import tpu as pltpu

_B, _H, _S, _D = 1, 12, 2048, 64
_BLK = 64
_SEL = 16
_NB = _S // _BLK
_TQ = 256


def _nsa_tc_kernel(bi_ref, q_ref, k_ref, v_ref, o_ref):
    qt = pl.program_id(1)
    q = q_ref[0]
    k = k_ref[0]
    v = v_ref[0]
    bi = bi_ref[0]

    scale = jnp.float32(1.0 / np.sqrt(_D))
    s = jax.lax.dot_general(
        q, k, (((1,), (1,)), ((), ())),
        preferred_element_type=jnp.float32) * scale

    qpos = qt * _TQ + jax.lax.broadcasted_iota(jnp.int32, (_TQ, 1), 0)
    kpos = jax.lax.broadcasted_iota(jnp.int32, (1, _S), 1)
    kblk = kpos // _BLK

    allowed = (kblk == qpos // _BLK)
    for j in range(_SEL):
        allowed = allowed | (bi[:, j:j + 1] == kblk)
    allowed = allowed & (kpos <= qpos)

    s = jnp.where(allowed, s, jnp.float32(-1e30))
    m = jnp.max(s, axis=-1, keepdims=True)
    e = jnp.exp(s - m)
    p = e / jnp.sum(e, axis=-1, keepdims=True)
    o_ref[0] = jax.lax.dot_general(
        p, v, (((1,), (0,)), ((), ())),
        preferred_element_type=jnp.float32)


def kernel(Q, K, V, BlockIndices):
    B, H, S, D = Q.shape
    q = Q.reshape(H, S, D)
    k = K.reshape(H, S, D)
    v = V.reshape(H, S, D)
    bi = BlockIndices.reshape(H, S, _SEL).astype(jnp.int32)

    grid = (H, S // _TQ)
    out = pl.pallas_call(
        _nsa_tc_kernel,
        grid=grid,
        in_specs=[
            pl.BlockSpec((1, _TQ, _SEL), lambda h, t: (h, t, 0)),
            pl.BlockSpec((1, _TQ, D), lambda h, t: (h, t, 0)),
            pl.BlockSpec((1, S, D), lambda h, t: (h, 0, 0)),
            pl.BlockSpec((1, S, D), lambda h, t: (h, 0, 0)),
        ],
        out_specs=pl.BlockSpec((1, _TQ, D), lambda h, t: (h, t, 0)),
        out_shape=jax.ShapeDtypeStruct((H, S, D), jnp.float32),
    )(bi, q, k, v)
    return out.reshape(B, H, S, D)


# speedup vs baseline: 2.0008x; 2.0008x over previous
import jax
import jax.numpy as jnp
import numpy as np
from jax.experimental import pallas as pl
from jax.experimental.pallas import tpu as pltpu

_B, _H, _S, _D = 1, 12, 2048, 64
_BLK = 64
_SEL = 16
_NB = _S // _BLK
_TQ = 256
_TK = 256


def _nsa_tc_kernel(bi_ref, q_ref, k_ref, v_ref, o_ref, s_scr):
    qt = pl.program_id(1)
    q = q_ref[0]
    bi = bi_ref[0]

    qpos = qt * _TQ + jax.lax.broadcasted_iota(jnp.int32, (_TQ, 1), 0)
    bits = jnp.left_shift(jnp.int32(1), qpos // _BLK)
    for j in range(_SEL):
        bits = bits | jnp.left_shift(jnp.int32(1), bi[:, j:j + 1])

    scale = jnp.float32(1.0 / np.sqrt(_D))
    n_kt = qt + 1

    def pass1(i, m):
        k = k_ref[0, pl.ds(i * _TK, _TK), :]
        s = jax.lax.dot_general(
            q, k, (((1,), (1,)), ((), ())),
            preferred_element_type=jnp.float32) * scale
        kpos = i * _TK + jax.lax.broadcasted_iota(jnp.int32, (1, _TK), 1)
        ok = (jnp.right_shift(bits, kpos // _BLK) & 1) != 0
        ok = ok & (kpos <= qpos)
        s = jnp.where(ok, s, jnp.float32(-1e30))
        s_scr[:, pl.ds(i * _TK, _TK)] = s
        return jnp.maximum(m, jnp.max(s, axis=-1, keepdims=True))

    m = jax.lax.fori_loop(
        0, n_kt, pass1, jnp.full((_TQ, 1), -1e30, jnp.float32))

    def pass2(i, carry):
        l, acc = carry
        e = jnp.exp(s_scr[:, pl.ds(i * _TK, _TK)] - m)
        v = v_ref[0, pl.ds(i * _TK, _TK), :]
        acc = acc + jax.lax.dot_general(
            e, v, (((1,), (0,)), ((), ())),
            preferred_element_type=jnp.float32)
        return l + jnp.sum(e, axis=-1, keepdims=True), acc

    l, acc = jax.lax.fori_loop(
        0, n_kt, pass2,
        (jnp.zeros((_TQ, 1), jnp.float32), jnp.zeros((_TQ, _D), jnp.float32)))

    o_ref[0] = acc / l


def kernel(Q, K, V, BlockIndices):
    B, H, S, D = Q.shape
    q = Q.reshape(H, S, D)
    k = K.reshape(H, S, D)
    v = V.reshape(H, S, D)
    bi = BlockIndices.reshape(H, S, _SEL).astype(jnp.int32)

    grid = (H, S // _TQ)
    out = pl.pallas_call(
        _nsa_tc_kernel,
        grid=grid,
        in_specs=[
            pl.BlockSpec((1, _TQ, _SEL), lambda h, t: (h, t, 0)),
            pl.BlockSpec((1, _TQ, D), lambda h, t: (h, t, 0)),
            pl.BlockSpec((1, S, D), lambda h, t: (h, 0, 0)),
            pl.BlockSpec((1, S, D), lambda h, t: (h, 0, 0)),
        ],
        out_specs=pl.BlockSpec((1, _TQ, D), lambda h, t: (h, t, 0)),
        out_shape=jax.ShapeDtypeStruct((H, S, D), jnp.float32),
        scratch_shapes=[pltpu.VMEM((_TQ, S), jnp.float32)],
    )(bi, q, k, v)
    return out.reshape(B, H, S, D)


# speedup vs baseline: 2.8147x; 1.4068x over previous
import jax
import jax.numpy as jnp
import numpy as np
from jax.experimental import pallas as pl
from jax.experimental.pallas import tpu as pltpu

_B, _H, _S, _D = 1, 12, 2048, 64
_BLK = 64
_SEL = 16
_NB = _S // _BLK
_TQ = 256
_TK = 512


def _nsa_tc_kernel(bi_ref, q_ref, k_ref, v_ref, o_ref):
    qt = pl.program_id(1)
    q = q_ref[0].astype(jnp.bfloat16)
    bi = bi_ref[0]

    qpos = qt * _TQ + jax.lax.broadcasted_iota(jnp.int32, (_TQ, 1), 0)
    t = jnp.left_shift(jnp.int32(1), bi)
    w = _SEL
    while w > 1:
        w //= 2
        t = t[:, :w] | t[:, w:2 * w]
    bits = t | jnp.left_shift(jnp.int32(1), qpos // _BLK)

    scale = jnp.float32(1.0 / np.sqrt(_D))
    n_kt = (qt * _TQ + _TQ + _TK - 1) // _TK

    def body(i, carry):
        m, l, acc = carry
        k = k_ref[0, pl.ds(i * _TK, _TK), :].astype(jnp.bfloat16)
        s = jax.lax.dot_general(
            q, k, (((1,), (1,)), ((), ())),
            preferred_element_type=jnp.float32) * scale
        kpos = i * _TK + jax.lax.broadcasted_iota(jnp.int32, (1, _TK), 1)
        ok = (jnp.right_shift(bits, kpos // _BLK) & 1) != 0
        ok = ok & (kpos <= qpos)
        s = jnp.where(ok, s, jnp.float32(-1e30))
        m_new = jnp.maximum(m, jnp.max(s, axis=-1, keepdims=True))
        alpha = jnp.exp(m - m_new)
        e = jnp.exp(s - m_new)
        v = v_ref[0, pl.ds(i * _TK, _TK), :].astype(jnp.bfloat16)
        acc = acc * alpha + jax.lax.dot_general(
            e.astype(jnp.bfloat16), v, (((1,), (0,)), ((), ())),
            preferred_element_type=jnp.float32)
        l = l * alpha + jnp.sum(e, axis=-1, keepdims=True)
        return m_new, l, acc

    m0 = jnp.full((_TQ, 1), -1e30, jnp.float32)
    l0 = jnp.zeros((_TQ, 1), jnp.float32)
    a0 = jnp.zeros((_TQ, _D), jnp.float32)
    _, l, acc = jax.lax.fori_loop(0, n_kt, body, (m0, l0, a0))

    o_ref[0] = acc / l


def kernel(Q, K, V, BlockIndices):
    B, H, S, D = Q.shape
    q = Q.reshape(H, S, D)
    k = K.reshape(H, S, D)
    v = V.reshape(H, S, D)
    bi = BlockIndices.reshape(H, S, _SEL).astype(jnp.int32)

    grid = (H, S // _TQ)
    out = pl.pallas_call(
        _nsa_tc_kernel,
        grid=grid,
        in_specs=[
            pl.BlockSpec((1, _TQ, _SEL), lambda h, t: (h, t, 0)),
            pl.BlockSpec((1, _TQ, D), lambda h, t: (h, t, 0)),
            pl.BlockSpec((1, S, D), lambda h, t: (h, 0, 0)),
            pl.BlockSpec((1, S, D), lambda h, t: (h, 0, 0)),
        ],
        out_specs=pl.BlockSpec((1, _TQ, D), lambda h, t: (h, t, 0)),
        out_shape=jax.ShapeDtypeStruct((H, S, D), jnp.float32),
    )(bi, q, k, v)
    return out.reshape(B, H, S, D)


# speedup vs baseline: 3.5594x; 1.2646x over previous
import jax
import jax.numpy as jnp
import numpy as np
from jax.experimental import pallas as pl
from jax.experimental.pallas import tpu as pltpu

_B, _H, _S, _D = 1, 12, 2048, 64
_BLK = 64
_SEL = 16
_NB = _S // _BLK
_TQ = 512
_TK = 512
_BPT = _TK // _BLK

_NEG = -1e30


def _nsa_tc_kernel(bi_ref, q_ref, k_ref, v_ref, o_ref):
    qt = pl.program_id(1)
    q = q_ref[0] * jnp.bfloat16(0.125)
    bi = bi_ref[0]
    e8 = (jax.lax.broadcasted_iota(jnp.int32, (_BPT, _TK), 1) // _BLK ==
          jax.lax.broadcasted_iota(jnp.int32, (_BPT, _TK), 0)
          ).astype(jnp.bfloat16)

    qpos = qt * _TQ + jax.lax.broadcasted_iota(jnp.int32, (_TQ, 1), 0)
    t = jnp.left_shift(jnp.int32(1), bi)
    w = _SEL
    while w > 1:
        w //= 2
        t = t[:, :w] | t[:, w:2 * w]
    bits = t | jnp.left_shift(jnp.int32(1), qpos // _BLK)

    b8 = jax.lax.broadcasted_iota(jnp.int32, (_TQ, _BPT), 1)

    def tile_bias(i):
        win = jax.lax.shift_right_logical(bits, i * _BPT)
        ok = (jax.lax.shift_right_logical(win, b8) & 1) != 0
        return jnp.where(ok, jnp.float32(0), jnp.float32(_NEG)
                         ).astype(jnp.bfloat16)

    def qk(k_slice):
        return jax.lax.dot_general(
            q, k_slice, (((1,), (1,)), ((), ())),
            preferred_element_type=jnp.float32)

    def expand(bias8):
        return jax.lax.dot_general(
            bias8, e8, (((1,), (0,)), ((), ())),
            preferred_element_type=jnp.float32)

    kd = k_ref[0, pl.ds(qt * _TK, _TK), :]
    vd = v_ref[0, pl.ds(qt * _TK, _TK), :]
    r_io = jax.lax.broadcasted_iota(jnp.int32, (_TQ, _TK), 0)
    c_io = jax.lax.broadcasted_iota(jnp.int32, (_TQ, _TK), 1)
    s = jnp.where(r_io >= c_io, qk(kd) + expand(tile_bias(qt)),
                  jnp.float32(_NEG))
    m = jnp.max(s, axis=-1, keepdims=True)
    e = jnp.exp(s - m)
    l = jnp.sum(e, axis=-1, keepdims=True)
    acc = jax.lax.dot_general(
        e.astype(jnp.bfloat16), vd, (((1,), (0,)), ((), ())),
        preferred_element_type=jnp.float32)

    def body(i, carry):
        m, l, acc = carry
        k = k_ref[0, pl.ds(i * _TK, _TK), :]
        s = qk(k) + expand(tile_bias(i))
        m_new = jnp.maximum(m, jnp.max(s, axis=-1, keepdims=True))
        alpha = jnp.exp(m - m_new)
        e = jnp.exp(s - m_new)
        v = v_ref[0, pl.ds(i * _TK, _TK), :]
        acc = acc * alpha + jax.lax.dot_general(
            e.astype(jnp.bfloat16), v, (((1,), (0,)), ((), ())),
            preferred_element_type=jnp.float32)
        l = l * alpha + jnp.sum(e, axis=-1, keepdims=True)
        return m_new, l, acc

    _, l, acc = jax.lax.fori_loop(0, qt, body, (m, l, acc))

    o_ref[0] = acc / l


def kernel(Q, K, V, BlockIndices):
    B, H, S, D = Q.shape
    q = Q.reshape(H, S, D).astype(jnp.bfloat16)
    k = K.reshape(H, S, D).astype(jnp.bfloat16)
    v = V.reshape(H, S, D).astype(jnp.bfloat16)
    bi = BlockIndices.reshape(H, S, _SEL).astype(jnp.int32)

    grid = (H, S // _TQ)
    out = pl.pallas_call(
        _nsa_tc_kernel,
        grid=grid,
        in_specs=[
            pl.BlockSpec((1, _TQ, _SEL), lambda h, t: (h, t, 0)),
            pl.BlockSpec((1, _TQ, D), lambda h, t: (h, t, 0)),
            pl.BlockSpec((1, S, D), lambda h, t: (h, 0, 0)),
            pl.BlockSpec((1, S, D), lambda h, t: (h, 0, 0)),
        ],
        out_specs=pl.BlockSpec((1, _TQ, D), lambda h, t: (h, t, 0)),
        out_shape=jax.ShapeDtypeStruct((H, S, D), jnp.float32),
    )(bi, q, k, v)
    return out.reshape(B, H, S, D)
